# baseline (speedup 1.0000x reference)
import functools

import jax
import jax.numpy as jnp
from jax import lax
from jax.experimental import pallas as pl
from jax.experimental.pallas import tpu as pltpu
from jax.experimental.pallas import tpu_sc as plsc

N = 10000
E = 320000
D = 128
H = 4
B = 10
NPG = N // B

NC = 2
NS = 16
NW = NC * NS
GP = 1024
NP = B * GP
GPH = GP * H
FH = NP * H
EPT = E // NW
CE = 2000
NCHUNK = EPT // CE

_mesh = plsc.VectorSubcoreMesh(core_axis_name="c", subcore_axis_name="s")


def _remap(n):
    g = (n.astype(jnp.float32) * jnp.float32(0.001)).astype(jnp.int32)
    return n + g * jnp.int32(GP - NPG)


@functools.partial(
    pl.kernel,
    out_type=jax.ShapeDtypeStruct((NW, NP), jnp.float32),
    mesh=_mesh,
    compiler_params=pltpu.CompilerParams(needs_layout_passes=False),
    scratch_types=[
        pltpu.VMEM((EPT,), jnp.int32),
        pltpu.VMEM((EPT,), jnp.float32),
        pltpu.VMEM((NP,), jnp.float32),
        pltpu.SemaphoreType.DMA,
        pltpu.SemaphoreType.DMA,
    ],
)
def _deg_kernel(eflat_hbm, w_hbm, out_hbm, col_v, w_v, deg_v, sem1, sem2):
    cid = lax.axis_index("c")
    sid = lax.axis_index("s")
    wid = cid * NS + sid
    base = wid * EPT
    cp1 = pltpu.async_copy(eflat_hbm.at[pl.ds(E + base, EPT)], col_v, sem1)
    cp2 = pltpu.async_copy(w_hbm.at[pl.ds(base, EPT)], w_v, sem2)

    zeros = jnp.zeros((16,), jnp.float32)

    def zero_i(i, _):
        for k in range(8):
            deg_v[pl.ds(i * 128 + k * 16, 16)] = zeros
        return 0

    lax.fori_loop(0, NP // 128, zero_i, 0)
    cp1.wait()
    cp2.wait()

    def edge_i(i, _):
        for k in range(5):
            s = pl.ds((i * 5 + k) * 16, 16)
            plsc.addupdate_scatter(deg_v, [_remap(col_v[s])], w_v[s])
        return 0

    lax.fori_loop(0, EPT // 80, edge_i, 0)
    pltpu.sync_copy(deg_v, out_hbm.at[wid])


def _lin_body(x_ref, w1_ref, xl0_ref, xl1_ref, xl2_ref, xl3_ref):
    xlt = lax.dot_general(w1_ref[...], x_ref[...],
                          dimension_numbers=(((0,), (1,)), ((), ())),
                          preferred_element_type=jnp.float32)
    ztail = jnp.zeros((GP - NPG,), jnp.float32)
    for c, ref in enumerate((xl0_ref, xl1_ref, xl2_ref, xl3_ref)):
        ref[:NPG] = xlt[c]
        ref[NPG:] = ztail


_lin_call = pl.pallas_call(
    _lin_body,
    grid=(B,),
    in_specs=[
        pl.BlockSpec((NPG, D), lambda g: (g, 0)),
        pl.BlockSpec((D, H), lambda g: (0, 0)),
    ],
    out_specs=(
        pl.BlockSpec((GP,), lambda g: (g,)),
        pl.BlockSpec((GP,), lambda g: (g,)),
        pl.BlockSpec((GP,), lambda g: (g,)),
        pl.BlockSpec((GP,), lambda g: (g,)),
    ),
    out_shape=(
        jax.ShapeDtypeStruct((NP,), jnp.float32),
        jax.ShapeDtypeStruct((NP,), jnp.float32),
        jax.ShapeDtypeStruct((NP,), jnp.float32),
        jax.ShapeDtypeStruct((NP,), jnp.float32),
    ),
)


def _dis_body(degp_ref, dis_ref):
    deg = jnp.sum(degp_ref[...], axis=0) + 1.0
    dis_ref[...] = lax.rsqrt(deg)


_dis_call = pl.pallas_call(
    _dis_body,
    out_shape=jax.ShapeDtypeStruct((NP,), jnp.float32),
)


@functools.partial(
    pl.kernel,
    out_type=jax.ShapeDtypeStruct((NW, FH), jnp.float32),
    mesh=_mesh,
    compiler_params=pltpu.CompilerParams(needs_layout_passes=False),
    scratch_types=[
        pltpu.VMEM((CE,), jnp.int32),
        pltpu.VMEM((CE,), jnp.int32),
        pltpu.VMEM((CE,), jnp.int32),
        pltpu.VMEM((CE,), jnp.int32),
        pltpu.VMEM((CE,), jnp.float32),
        pltpu.VMEM((CE,), jnp.float32),
        pltpu.VMEM((FH,), jnp.float32),
        pltpu.VMEM((NP,), jnp.float32),
        pltpu.VMEM((FH,), jnp.float32),
        pltpu.SemaphoreType.DMA,
        pltpu.SemaphoreType.DMA,
        pltpu.SemaphoreType.DMA,
        pltpu.SemaphoreType.DMA,
        pltpu.SemaphoreType.DMA,
    ],
)
def _msg_kernel(eflat_hbm, w_hbm, xl0_hbm, xl1_hbm, xl2_hbm, xl3_hbm,
                dis_hbm, out_hbm,
                row_v0, row_v1, col_v0, col_v1, w_v0, w_v1, xl_v, dis_v, h_v,
                semr, semc, semw, semx, semd):
    row_b = (row_v0, row_v1)
    col_b = (col_v0, col_v1)
    w_b = (w_v0, w_v1)
    cid = lax.axis_index("c")
    sid = lax.axis_index("s")
    wid = cid * NS + sid
    base = wid * EPT

    cpx = [
        pltpu.async_copy(plane, xl_v.at[pl.ds(c * NP, NP)], semx)
        for c, plane in enumerate((xl0_hbm, xl1_hbm, xl2_hbm, xl3_hbm))
    ]
    cpd = pltpu.async_copy(dis_hbm, dis_v, semd)

    def issue(ck, buf):
        cb = base + ck * CE
        return (
            pltpu.async_copy(eflat_hbm.at[pl.ds(cb, CE)], row_b[buf], semr),
            pltpu.async_copy(eflat_hbm.at[pl.ds(E + cb, CE)], col_b[buf], semc),
            pltpu.async_copy(w_hbm.at[pl.ds(cb, CE)], w_b[buf], semw),
        )

    pend = issue(0, 0)

    zeros = jnp.zeros((16,), jnp.float32)

    def zero_i(i, _):
        for k in range(8):
            h_v[pl.ds(i * 128 + k * 16, 16)] = zeros
        return 0

    lax.fori_loop(0, FH // 128, zero_i, 0)
    for cp in cpx:
        cp.wait()
    cpd.wait()

    four = jnp.int32(4)

    def make_edge_body(buf):
        rv, cv, wvr = row_b[buf], col_b[buf], w_b[buf]

        def edge_i(i, _):
            for k in range(25):
                s = pl.ds((i * 25 + k) * 16, 16)
                r = _remap(rv[s])
                c = _remap(cv[s])
                wv = wvr[s]
                dr = plsc.load_gather(dis_v, [r])
                dc = plsc.load_gather(dis_v, [c])
                nrm = dr * wv * dc
                cb = c * four
                for comp in range(H):
                    xv = plsc.load_gather(xl_v, [r + jnp.int32(comp * NP)])
                    plsc.addupdate_scatter(h_v, [cb + comp], xv * nrm)
            return 0
        return edge_i

    for ck in range(NCHUNK):
        for cp in pend:
            cp.wait()
        if ck + 1 < NCHUNK:
            nxt = issue(ck + 1, (ck + 1) % 2)
        lax.fori_loop(0, CE // 400, make_edge_body(ck % 2), 0)
        if ck + 1 < NCHUNK:
            pend = nxt

    fb = sid * (FH // NS)
    iota = lax.iota(jnp.int32, 16)
    three = jnp.int32(3)

    @pl.when(cid == 0)
    def _selfloop():
        def loop_j(j, _):
            off = fb + j * 16
            flat = off + iota
            node = flat >> 2
            comp = flat & three
            d = plsc.load_gather(dis_v, [node])
            xv = plsc.load_gather(xl_v, [node + comp * jnp.int32(NP)])
            s = pl.ds(off, 16)
            h_v[s] = h_v[s] + d * d * xv
            return 0

        lax.fori_loop(0, FH // NS // 16, loop_j, 0)

    pltpu.sync_copy(h_v, out_hbm.at[wid])


def _head_body(hp_ref, b1_ref, wfc1_ref, bfc1_ref, wfc2_ref, bfc2_ref,
               out_ref, wp_ref):
    wp_ref[:NPG * H, :] = wfc1_ref[...]
    wp_ref[NPG * H:, :] = jnp.zeros((GPH - NPG * H, 32), jnp.float32)
    wsum = jnp.sum(wfc1_ref[...].reshape(NPG, H, 32), axis=0)
    beff = (jnp.dot(b1_ref[...], wsum, preferred_element_type=jnp.float32)
            + bfc1_ref[...])

    h = jnp.concatenate(
        [jnp.sum(hp_ref[:, g * GPH:(g + 1) * GPH], axis=0, keepdims=True)
         for g in range(B)], axis=0)
    a = jnp.dot(h, wp_ref[...], preferred_element_type=jnp.float32)
    a = jnp.maximum(a + beff, 0.0)
    out_ref[...] = jnp.dot(a, wfc2_ref[...],
                           preferred_element_type=jnp.float32) + bfc2_ref[...]


_head_call = pl.pallas_call(
    _head_body,
    out_shape=jax.ShapeDtypeStruct((B, 2), jnp.float32),
    scratch_shapes=[
        pltpu.VMEM((GPH, 32), jnp.float32),
    ],
)


def kernel(x, edge_index, edge_attr, y, adj, W1, b1, Wfc1, bfc1, Wfc2, bfc2):
    eflat = edge_index.reshape(2 * E)
    deg_p = _deg_kernel(eflat, edge_attr)
    xl0, xl1, xl2, xl3 = _lin_call(x, W1)
    dis = _dis_call(deg_p)
    h_p = _msg_kernel(eflat, edge_attr, xl0, xl1, xl2, xl3, dis)

    logits = _head_call(h_p, b1.reshape(1, H), Wfc1, bfc1.reshape(1, 32),
                        Wfc2, bfc2.reshape(1, 2))
    reg = jnp.zeros((1,), dtype=jnp.float32)
    return (logits, reg)


# speedup vs baseline: 1.3668x; 1.3668x over previous
import functools

import jax
import jax.numpy as jnp
from jax import lax
from jax.experimental import pallas as pl
from jax.experimental.pallas import tpu as pltpu
from jax.experimental.pallas import tpu_sc as plsc

N = 10000
E = 320000
D = 128
H = 4
B = 10
NPG = N // B

NC = 2
NS = 16
NW = NC * NS
GP = 1024
NP = B * GP
GPH = GP * H
FH = NP * H
EPT = E // NW
CE = 2000
NCHUNK = EPT // CE

_mesh = plsc.VectorSubcoreMesh(core_axis_name="c", subcore_axis_name="s")


def _remap(n):
    g = (n.astype(jnp.float32) * jnp.float32(0.001)).astype(jnp.int32)
    return n + g * jnp.int32(GP - NPG)


@functools.partial(
    pl.kernel,
    out_type=jax.ShapeDtypeStruct((NW, NP), jnp.float32),
    mesh=_mesh,
    compiler_params=pltpu.CompilerParams(needs_layout_passes=False),
    scratch_types=[
        pltpu.VMEM((EPT,), jnp.int32),
        pltpu.VMEM((EPT,), jnp.float32),
        pltpu.VMEM((NP,), jnp.float32),
        pltpu.SemaphoreType.DMA,
        pltpu.SemaphoreType.DMA,
    ],
)
def _deg_kernel(eflat_hbm, w_hbm, out_hbm, col_v, w_v, deg_v, sem1, sem2):
    cid = lax.axis_index("c")
    sid = lax.axis_index("s")
    wid = cid * NS + sid
    base = wid * EPT
    cp1 = pltpu.async_copy(eflat_hbm.at[pl.ds(E + base, EPT)], col_v, sem1)
    cp2 = pltpu.async_copy(w_hbm.at[pl.ds(base, EPT)], w_v, sem2)

    zeros = jnp.zeros((16,), jnp.float32)

    def zero_i(i, _):
        for k in range(8):
            deg_v[pl.ds(i * 128 + k * 16, 16)] = zeros
        return 0

    lax.fori_loop(0, NP // 128, zero_i, 0)
    cp1.wait()
    cp2.wait()

    def edge_i(i, _):
        for k in range(5):
            s = pl.ds((i * 5 + k) * 16, 16)
            plsc.addupdate_scatter(deg_v, [_remap(col_v[s])], w_v[s])
        return 0

    lax.fori_loop(0, EPT // 80, edge_i, 0)
    pltpu.sync_copy(deg_v, out_hbm.at[wid])


def _lin_body(x_ref, w1_ref, xl0_ref, xl1_ref, xl2_ref, xl3_ref):
    xlt = lax.dot_general(w1_ref[...], x_ref[...],
                          dimension_numbers=(((0,), (1,)), ((), ())),
                          preferred_element_type=jnp.float32)
    ztail = jnp.zeros((GP - NPG,), jnp.float32)
    for c, ref in enumerate((xl0_ref, xl1_ref, xl2_ref, xl3_ref)):
        ref[:NPG] = xlt[c]
        ref[NPG:] = ztail


_lin_call = pl.pallas_call(
    _lin_body,
    grid=(B,),
    in_specs=[
        pl.BlockSpec((NPG, D), lambda g: (g, 0)),
        pl.BlockSpec((D, H), lambda g: (0, 0)),
    ],
    out_specs=(
        pl.BlockSpec((GP,), lambda g: (g,)),
        pl.BlockSpec((GP,), lambda g: (g,)),
        pl.BlockSpec((GP,), lambda g: (g,)),
        pl.BlockSpec((GP,), lambda g: (g,)),
    ),
    out_shape=(
        jax.ShapeDtypeStruct((NP,), jnp.float32),
        jax.ShapeDtypeStruct((NP,), jnp.float32),
        jax.ShapeDtypeStruct((NP,), jnp.float32),
        jax.ShapeDtypeStruct((NP,), jnp.float32),
    ),
)


def _dis_body(degp_ref, dis_ref):
    deg = jnp.sum(degp_ref[...], axis=0) + 1.0
    dis_ref[...] = lax.rsqrt(deg)


_dis_call = pl.pallas_call(
    _dis_body,
    out_shape=jax.ShapeDtypeStruct((NP,), jnp.float32),
)


@functools.partial(
    pl.kernel,
    out_type=jax.ShapeDtypeStruct((NW, FH), jnp.float32),
    mesh=_mesh,
    compiler_params=pltpu.CompilerParams(needs_layout_passes=False),
    scratch_types=[
        pltpu.VMEM((CE,), jnp.int32),
        pltpu.VMEM((CE,), jnp.int32),
        pltpu.VMEM((CE,), jnp.int32),
        pltpu.VMEM((CE,), jnp.int32),
        pltpu.VMEM((CE,), jnp.float32),
        pltpu.VMEM((CE,), jnp.float32),
        pltpu.VMEM((FH,), jnp.float32),
        pltpu.VMEM((NP,), jnp.float32),
        pltpu.VMEM((FH,), jnp.float32),
        pltpu.SemaphoreType.DMA,
        pltpu.SemaphoreType.DMA,
        pltpu.SemaphoreType.DMA,
        pltpu.SemaphoreType.DMA,
        pltpu.SemaphoreType.DMA,
    ],
)
def _msg_kernel(eflat_hbm, w_hbm, xl0_hbm, xl1_hbm, xl2_hbm, xl3_hbm,
                dis_hbm, out_hbm,
                row_v0, row_v1, col_v0, col_v1, w_v0, w_v1, xl_v, dis_v, h_v,
                semr, semc, semw, semx, semd):
    row_b = (row_v0, row_v1)
    col_b = (col_v0, col_v1)
    w_b = (w_v0, w_v1)
    cid = lax.axis_index("c")
    sid = lax.axis_index("s")
    wid = cid * NS + sid
    base = wid * EPT

    cpx = [
        pltpu.async_copy(plane, xl_v.at[pl.ds(c * NP, NP)], semx)
        for c, plane in enumerate((xl0_hbm, xl1_hbm, xl2_hbm, xl3_hbm))
    ]
    cpd = pltpu.async_copy(dis_hbm, dis_v, semd)

    def issue(ck, buf):
        cb = base + ck * CE
        return (
            pltpu.async_copy(eflat_hbm.at[pl.ds(cb, CE)], row_b[buf], semr),
            pltpu.async_copy(eflat_hbm.at[pl.ds(E + cb, CE)], col_b[buf], semc),
            pltpu.async_copy(w_hbm.at[pl.ds(cb, CE)], w_b[buf], semw),
        )

    pend = issue(0, 0)

    zeros = jnp.zeros((16,), jnp.float32)

    def zero_i(i, _):
        for k in range(8):
            h_v[pl.ds(i * 128 + k * 16, 16)] = zeros
        return 0

    lax.fori_loop(0, FH // 128, zero_i, 0)
    for cp in cpx:
        cp.wait()
    cpd.wait()

    four = jnp.int32(4)

    def run_edges(buf):
        rv, cv, wvr = row_b[buf], col_b[buf], w_b[buf]

        @plsc.parallel_loop(0, CE // 16, 1, unroll=4)
        def _edge_i(i):
            s = pl.ds(i * 16, 16)
            r = _remap(rv[s])
            c = _remap(cv[s])
            wv = wvr[s]
            dr = plsc.load_gather(dis_v, [r])
            dc = plsc.load_gather(dis_v, [c])
            nrm = dr * wv * dc
            cb = c * four
            for comp in range(H):
                xv = plsc.load_gather(xl_v, [r + jnp.int32(comp * NP)])
                plsc.addupdate_scatter(h_v, [cb + comp], xv * nrm)

    for ck in range(NCHUNK):
        for cp in pend:
            cp.wait()
        if ck + 1 < NCHUNK:
            nxt = issue(ck + 1, (ck + 1) % 2)
        run_edges(ck % 2)
        if ck + 1 < NCHUNK:
            pend = nxt

    fb = sid * (FH // NS)
    iota = lax.iota(jnp.int32, 16)
    three = jnp.int32(3)

    @pl.when(cid == 0)
    def _selfloop():
        def loop_j(j, _):
            off = fb + j * 16
            flat = off + iota
            node = flat >> 2
            comp = flat & three
            d = plsc.load_gather(dis_v, [node])
            xv = plsc.load_gather(xl_v, [node + comp * jnp.int32(NP)])
            s = pl.ds(off, 16)
            h_v[s] = h_v[s] + d * d * xv
            return 0

        lax.fori_loop(0, FH // NS // 16, loop_j, 0)

    pltpu.sync_copy(h_v, out_hbm.at[wid])


def _head_body(hp_ref, b1_ref, wfc1_ref, bfc1_ref, wfc2_ref, bfc2_ref,
               out_ref, wp_ref):
    wp_ref[:NPG * H, :] = wfc1_ref[...]
    wp_ref[NPG * H:, :] = jnp.zeros((GPH - NPG * H, 32), jnp.float32)
    wsum = jnp.sum(wfc1_ref[...].reshape(NPG, H, 32), axis=0)
    beff = (jnp.dot(b1_ref[...], wsum, preferred_element_type=jnp.float32)
            + bfc1_ref[...])

    h = jnp.concatenate(
        [jnp.sum(hp_ref[:, g * GPH:(g + 1) * GPH], axis=0, keepdims=True)
         for g in range(B)], axis=0)
    a = jnp.dot(h, wp_ref[...], preferred_element_type=jnp.float32)
    a = jnp.maximum(a + beff, 0.0)
    out_ref[...] = jnp.dot(a, wfc2_ref[...],
                           preferred_element_type=jnp.float32) + bfc2_ref[...]


_head_call = pl.pallas_call(
    _head_body,
    out_shape=jax.ShapeDtypeStruct((B, 2), jnp.float32),
    scratch_shapes=[
        pltpu.VMEM((GPH, 32), jnp.float32),
    ],
)


def kernel(x, edge_index, edge_attr, y, adj, W1, b1, Wfc1, bfc1, Wfc2, bfc2):
    eflat = edge_index.reshape(2 * E)
    deg_p = _deg_kernel(eflat, edge_attr)
    xl0, xl1, xl2, xl3 = _lin_call(x, W1)
    dis = _dis_call(deg_p)
    h_p = _msg_kernel(eflat, edge_attr, xl0, xl1, xl2, xl3, dis)

    logits = _head_call(h_p, b1.reshape(1, H), Wfc1, bfc1.reshape(1, 32),
                        Wfc2, bfc2.reshape(1, 2))
    reg = jnp.zeros((1,), dtype=jnp.float32)
    return (logits, reg)


# speedup vs baseline: 1.4574x; 1.0663x over previous
import functools

import jax
import jax.numpy as jnp
from jax import lax
from jax.experimental import pallas as pl
from jax.experimental.pallas import tpu as pltpu
from jax.experimental.pallas import tpu_sc as plsc

N = 10000
E = 320000
D = 128
H = 4
B = 10
NPG = N // B

NC = 2
NS = 16
NW = NC * NS
GP = 1024
NP = B * GP
GPH = GP * H
FH = NP * H
EPT = E // NW
CE = 2000
NCHUNK = EPT // CE

_mesh = plsc.VectorSubcoreMesh(core_axis_name="c", subcore_axis_name="s")


def _remap(n):
    g = (n.astype(jnp.float32) * jnp.float32(0.001)).astype(jnp.int32)
    return n + g * jnp.int32(GP - NPG)


@functools.partial(
    pl.kernel,
    out_type=jax.ShapeDtypeStruct((NW, NP), jnp.float32),
    mesh=_mesh,
    compiler_params=pltpu.CompilerParams(needs_layout_passes=False),
    scratch_types=[
        pltpu.VMEM((EPT,), jnp.int32),
        pltpu.VMEM((EPT,), jnp.float32),
        pltpu.VMEM((NP,), jnp.float32),
        pltpu.SemaphoreType.DMA,
        pltpu.SemaphoreType.DMA,
    ],
)
def _deg_kernel(eflat_hbm, w_hbm, out_hbm, col_v, w_v, deg_v, sem1, sem2):
    cid = lax.axis_index("c")
    sid = lax.axis_index("s")
    wid = cid * NS + sid
    base = wid * EPT
    cp1 = pltpu.async_copy(eflat_hbm.at[pl.ds(E + base, EPT)], col_v, sem1)
    cp2 = pltpu.async_copy(w_hbm.at[pl.ds(base, EPT)], w_v, sem2)

    zeros = jnp.zeros((16,), jnp.float32)

    @plsc.parallel_loop(0, NP // 16, 1, unroll=8)
    def _zero_i(i):
        deg_v[pl.ds(i * 16, 16)] = zeros

    cp1.wait()
    cp2.wait()

    @plsc.parallel_loop(0, EPT // 16, 1, unroll=4)
    def _edge_i(i):
        s = pl.ds(i * 16, 16)
        plsc.addupdate_scatter(deg_v, [_remap(col_v[s])], w_v[s])

    pltpu.sync_copy(deg_v, out_hbm.at[wid])


def _lin_body(x_ref, w1_ref, xl0_ref, xl1_ref, xl2_ref, xl3_ref):
    xlt = lax.dot_general(w1_ref[...], x_ref[...],
                          dimension_numbers=(((0,), (1,)), ((), ())),
                          preferred_element_type=jnp.float32)
    ztail = jnp.zeros((GP - NPG,), jnp.float32)
    for c, ref in enumerate((xl0_ref, xl1_ref, xl2_ref, xl3_ref)):
        ref[:NPG] = xlt[c]
        ref[NPG:] = ztail


_lin_call = pl.pallas_call(
    _lin_body,
    grid=(B,),
    in_specs=[
        pl.BlockSpec((NPG, D), lambda g: (g, 0)),
        pl.BlockSpec((D, H), lambda g: (0, 0)),
    ],
    out_specs=(
        pl.BlockSpec((GP,), lambda g: (g,)),
        pl.BlockSpec((GP,), lambda g: (g,)),
        pl.BlockSpec((GP,), lambda g: (g,)),
        pl.BlockSpec((GP,), lambda g: (g,)),
    ),
    out_shape=(
        jax.ShapeDtypeStruct((NP,), jnp.float32),
        jax.ShapeDtypeStruct((NP,), jnp.float32),
        jax.ShapeDtypeStruct((NP,), jnp.float32),
        jax.ShapeDtypeStruct((NP,), jnp.float32),
    ),
)


def _dis_body(degp_ref, dis_ref):
    deg = jnp.sum(degp_ref[...], axis=0) + 1.0
    dis_ref[...] = lax.rsqrt(deg)


_dis_call = pl.pallas_call(
    _dis_body,
    out_shape=jax.ShapeDtypeStruct((NP,), jnp.float32),
)


@functools.partial(
    pl.kernel,
    out_type=jax.ShapeDtypeStruct((NW, FH), jnp.float32),
    mesh=_mesh,
    compiler_params=pltpu.CompilerParams(needs_layout_passes=False),
    scratch_types=[
        pltpu.VMEM((CE,), jnp.int32),
        pltpu.VMEM((CE,), jnp.int32),
        pltpu.VMEM((CE,), jnp.int32),
        pltpu.VMEM((CE,), jnp.int32),
        pltpu.VMEM((CE,), jnp.float32),
        pltpu.VMEM((CE,), jnp.float32),
        pltpu.VMEM((FH,), jnp.float32),
        pltpu.VMEM((NP,), jnp.float32),
        pltpu.VMEM((FH,), jnp.float32),
        pltpu.SemaphoreType.DMA,
        pltpu.SemaphoreType.DMA,
        pltpu.SemaphoreType.DMA,
        pltpu.SemaphoreType.DMA,
        pltpu.SemaphoreType.DMA,
    ],
)
def _msg_kernel(eflat_hbm, w_hbm, xl0_hbm, xl1_hbm, xl2_hbm, xl3_hbm,
                dis_hbm, out_hbm,
                row_v0, row_v1, col_v0, col_v1, w_v0, w_v1, xl_v, dis_v, h_v,
                semr, semc, semw, semx, semd):
    row_b = (row_v0, row_v1)
    col_b = (col_v0, col_v1)
    w_b = (w_v0, w_v1)
    cid = lax.axis_index("c")
    sid = lax.axis_index("s")
    wid = cid * NS + sid
    base = wid * EPT

    cpx = [
        pltpu.async_copy(plane, xl_v.at[pl.ds(c * NP, NP)], semx)
        for c, plane in enumerate((xl0_hbm, xl1_hbm, xl2_hbm, xl3_hbm))
    ]
    cpd = pltpu.async_copy(dis_hbm, dis_v, semd)

    def issue(ck, buf):
        cb = base + ck * CE
        return (
            pltpu.async_copy(eflat_hbm.at[pl.ds(cb, CE)], row_b[buf], semr),
            pltpu.async_copy(eflat_hbm.at[pl.ds(E + cb, CE)], col_b[buf], semc),
            pltpu.async_copy(w_hbm.at[pl.ds(cb, CE)], w_b[buf], semw),
        )

    pend = issue(0, 0)

    zeros = jnp.zeros((16,), jnp.float32)

    @plsc.parallel_loop(0, FH // 16, 1, unroll=8)
    def _zero_i(i):
        h_v[pl.ds(i * 16, 16)] = zeros

    for cp in cpx:
        cp.wait()
    cpd.wait()

    four = jnp.int32(4)

    def run_edges(buf):
        rv, cv, wvr = row_b[buf], col_b[buf], w_b[buf]

        @plsc.parallel_loop(0, CE // 16, 1, unroll=4)
        def _edge_i(i):
            s = pl.ds(i * 16, 16)
            r = _remap(rv[s])
            c = _remap(cv[s])
            wv = wvr[s]
            dr = plsc.load_gather(dis_v, [r])
            dc = plsc.load_gather(dis_v, [c])
            nrm = dr * wv * dc
            cb = c * four
            for comp in range(H):
                xv = plsc.load_gather(xl_v, [r + jnp.int32(comp * NP)])
                plsc.addupdate_scatter(h_v, [cb + comp], xv * nrm)

    for ck in range(NCHUNK):
        for cp in pend:
            cp.wait()
        if ck + 1 < NCHUNK:
            nxt = issue(ck + 1, (ck + 1) % 2)
        run_edges(ck % 2)
        if ck + 1 < NCHUNK:
            pend = nxt

    fb = sid * (FH // NS)
    iota = lax.iota(jnp.int32, 16)
    three = jnp.int32(3)

    @pl.when(cid == 0)
    def _selfloop():
        @plsc.parallel_loop(0, FH // NS // 16, 1, unroll=4)
        def _loop_j(j):
            off = fb + j * 16
            flat = off + iota
            node = flat >> 2
            comp = flat & three
            d = plsc.load_gather(dis_v, [node])
            xv = plsc.load_gather(xl_v, [node + comp * jnp.int32(NP)])
            s = pl.ds(off, 16)
            h_v[s] = h_v[s] + d * d * xv

    pltpu.sync_copy(h_v, out_hbm.at[wid])


def _head_body(hp_ref, b1_ref, wfc1_ref, bfc1_ref, wfc2_ref, bfc2_ref,
               out_ref, wp_ref):
    wp_ref[:NPG * H, :] = wfc1_ref[...]
    wp_ref[NPG * H:, :] = jnp.zeros((GPH - NPG * H, 32), jnp.float32)
    wsum = jnp.sum(wfc1_ref[...].reshape(NPG, H, 32), axis=0)
    beff = (jnp.dot(b1_ref[...], wsum, preferred_element_type=jnp.float32)
            + bfc1_ref[...])

    h = jnp.concatenate(
        [jnp.sum(hp_ref[:, g * GPH:(g + 1) * GPH], axis=0, keepdims=True)
         for g in range(B)], axis=0)
    a = jnp.dot(h, wp_ref[...], preferred_element_type=jnp.float32)
    a = jnp.maximum(a + beff, 0.0)
    out_ref[...] = jnp.dot(a, wfc2_ref[...],
                           preferred_element_type=jnp.float32) + bfc2_ref[...]


_head_call = pl.pallas_call(
    _head_body,
    out_shape=jax.ShapeDtypeStruct((B, 2), jnp.float32),
    scratch_shapes=[
        pltpu.VMEM((GPH, 32), jnp.float32),
    ],
)


def kernel(x, edge_index, edge_attr, y, adj, W1, b1, Wfc1, bfc1, Wfc2, bfc2):
    eflat = edge_index.reshape(2 * E)
    deg_p = _deg_kernel(eflat, edge_attr)
    xl0, xl1, xl2, xl3 = _lin_call(x, W1)
    dis = _dis_call(deg_p)
    h_p = _msg_kernel(eflat, edge_attr, xl0, xl1, xl2, xl3, dis)

    logits = _head_call(h_p, b1.reshape(1, H), Wfc1, bfc1.reshape(1, 32),
                        Wfc2, bfc2.reshape(1, 2))
    reg = jnp.zeros((1,), dtype=jnp.float32)
    return (logits, reg)


# speedup vs baseline: 1.5262x; 1.0472x over previous
import functools

import jax
import jax.numpy as jnp
from jax import lax
from jax.experimental import pallas as pl
from jax.experimental.pallas import tpu as pltpu
from jax.experimental.pallas import tpu_sc as plsc

N = 10000
E = 320000
D = 128
H = 4
B = 10
NPG = N // B

NC = 2
NS = 16
NW = NC * NS
GP = 1024
NP = B * GP
GPH = GP * H
FH = NP * H
EPT = E // NW
CE = 2000
NCHUNK = EPT // CE

_mesh = plsc.VectorSubcoreMesh(core_axis_name="c", subcore_axis_name="s")


def _remap(n):
    g = (n.astype(jnp.float32) * jnp.float32(0.001)).astype(jnp.int32)
    return n + g * jnp.int32(GP - NPG)


@functools.partial(
    pl.kernel,
    out_type=jax.ShapeDtypeStruct((NW, NP), jnp.float32),
    mesh=_mesh,
    compiler_params=pltpu.CompilerParams(needs_layout_passes=False),
    scratch_types=[
        pltpu.VMEM((EPT,), jnp.int32),
        pltpu.VMEM((EPT,), jnp.float32),
        pltpu.VMEM((NP,), jnp.float32),
        pltpu.SemaphoreType.DMA,
        pltpu.SemaphoreType.DMA,
    ],
)
def _deg_kernel(eflat_hbm, w_hbm, out_hbm, col_v, w_v, deg_v, sem1, sem2):
    cid = lax.axis_index("c")
    sid = lax.axis_index("s")
    wid = cid * NS + sid
    base = wid * EPT
    cp1 = pltpu.async_copy(eflat_hbm.at[pl.ds(E + base, EPT)], col_v, sem1)
    cp2 = pltpu.async_copy(w_hbm.at[pl.ds(base, EPT)], w_v, sem2)

    zeros = jnp.zeros((16,), jnp.float32)

    @plsc.parallel_loop(0, NP // 16, 1, unroll=8)
    def _zero_i(i):
        deg_v[pl.ds(i * 16, 16)] = zeros

    cp1.wait()
    cp2.wait()

    @plsc.parallel_loop(0, EPT // 16, 1, unroll=4)
    def _edge_i(i):
        s = pl.ds(i * 16, 16)
        plsc.addupdate_scatter(deg_v, [_remap(col_v[s])], w_v[s])

    pltpu.sync_copy(deg_v, out_hbm.at[wid])


def _lin_body(x_ref, w1_ref, xl0_ref, xl1_ref, xl2_ref, xl3_ref):
    xlt = lax.dot_general(w1_ref[...], x_ref[...],
                          dimension_numbers=(((0,), (1,)), ((), ())),
                          preferred_element_type=jnp.float32)
    ztail = jnp.zeros((GP - NPG,), jnp.float32)
    for c, ref in enumerate((xl0_ref, xl1_ref, xl2_ref, xl3_ref)):
        ref[:NPG] = xlt[c, :NPG]
        ref[NPG:GP] = ztail
        ref[GP:GP + NPG] = xlt[c, NPG:]
        ref[GP + NPG:] = ztail


_lin_call = pl.pallas_call(
    _lin_body,
    grid=(B // 2,),
    in_specs=[
        pl.BlockSpec((2 * NPG, D), lambda g: (g, 0)),
        pl.BlockSpec((D, H), lambda g: (0, 0)),
    ],
    out_specs=(
        pl.BlockSpec((2 * GP,), lambda g: (g,)),
        pl.BlockSpec((2 * GP,), lambda g: (g,)),
        pl.BlockSpec((2 * GP,), lambda g: (g,)),
        pl.BlockSpec((2 * GP,), lambda g: (g,)),
    ),
    out_shape=(
        jax.ShapeDtypeStruct((NP,), jnp.float32),
        jax.ShapeDtypeStruct((NP,), jnp.float32),
        jax.ShapeDtypeStruct((NP,), jnp.float32),
        jax.ShapeDtypeStruct((NP,), jnp.float32),
    ),
)


def _dis_body(degp_ref, dis_ref):
    deg = jnp.sum(degp_ref[...], axis=0) + 1.0
    dis_ref[...] = lax.rsqrt(deg)


_dis_call = pl.pallas_call(
    _dis_body,
    out_shape=jax.ShapeDtypeStruct((NP,), jnp.float32),
)


@functools.partial(
    pl.kernel,
    out_type=jax.ShapeDtypeStruct((NW, FH), jnp.float32),
    mesh=_mesh,
    compiler_params=pltpu.CompilerParams(needs_layout_passes=False),
    scratch_types=[
        pltpu.VMEM((CE,), jnp.int32),
        pltpu.VMEM((CE,), jnp.int32),
        pltpu.VMEM((CE,), jnp.int32),
        pltpu.VMEM((CE,), jnp.int32),
        pltpu.VMEM((CE,), jnp.float32),
        pltpu.VMEM((CE,), jnp.float32),
        pltpu.VMEM((FH,), jnp.float32),
        pltpu.VMEM((NP,), jnp.float32),
        pltpu.VMEM((FH,), jnp.float32),
        pltpu.SemaphoreType.DMA,
        pltpu.SemaphoreType.DMA,
        pltpu.SemaphoreType.DMA,
        pltpu.SemaphoreType.DMA,
        pltpu.SemaphoreType.DMA,
    ],
)
def _msg_kernel(eflat_hbm, w_hbm, xl0_hbm, xl1_hbm, xl2_hbm, xl3_hbm,
                dis_hbm, out_hbm,
                row_v0, row_v1, col_v0, col_v1, w_v0, w_v1, xl_v, dis_v, h_v,
                semr, semc, semw, semx, semd):
    row_b = (row_v0, row_v1)
    col_b = (col_v0, col_v1)
    w_b = (w_v0, w_v1)
    cid = lax.axis_index("c")
    sid = lax.axis_index("s")
    wid = cid * NS + sid
    base = wid * EPT

    cpx = [
        pltpu.async_copy(plane, xl_v.at[pl.ds(c * NP, NP)], semx)
        for c, plane in enumerate((xl0_hbm, xl1_hbm, xl2_hbm, xl3_hbm))
    ]
    cpd = pltpu.async_copy(dis_hbm, dis_v, semd)

    def issue(ck, buf):
        cb = base + ck * CE
        return (
            pltpu.async_copy(eflat_hbm.at[pl.ds(cb, CE)], row_b[buf], semr),
            pltpu.async_copy(eflat_hbm.at[pl.ds(E + cb, CE)], col_b[buf], semc),
            pltpu.async_copy(w_hbm.at[pl.ds(cb, CE)], w_b[buf], semw),
        )

    pend = issue(0, 0)

    zeros = jnp.zeros((16,), jnp.float32)

    @plsc.parallel_loop(0, FH // 16, 1, unroll=8)
    def _zero_i(i):
        h_v[pl.ds(i * 16, 16)] = zeros

    for cp in cpx:
        cp.wait()
    cpd.wait()

    four = jnp.int32(4)

    def run_edges(buf):
        rv, cv, wvr = row_b[buf], col_b[buf], w_b[buf]

        @plsc.parallel_loop(0, CE // 16, 1, unroll=4)
        def _edge_i(i):
            s = pl.ds(i * 16, 16)
            r = _remap(rv[s])
            c = _remap(cv[s])
            wv = wvr[s]
            dr = plsc.load_gather(dis_v, [r])
            dc = plsc.load_gather(dis_v, [c])
            nrm = dr * wv * dc
            cb = c * four
            for comp in range(H):
                xv = plsc.load_gather(xl_v, [r + jnp.int32(comp * NP)])
                plsc.addupdate_scatter(h_v, [cb + comp], xv * nrm)

    for ck in range(NCHUNK):
        for cp in pend:
            cp.wait()
        if ck + 1 < NCHUNK:
            nxt = issue(ck + 1, (ck + 1) % 2)
        run_edges(ck % 2)
        if ck + 1 < NCHUNK:
            pend = nxt

    fb = sid * (FH // NS)
    iota = lax.iota(jnp.int32, 16)
    three = jnp.int32(3)

    @pl.when(cid == 0)
    def _selfloop():
        @plsc.parallel_loop(0, FH // NS // 16, 1, unroll=4)
        def _loop_j(j):
            off = fb + j * 16
            flat = off + iota
            node = flat >> 2
            comp = flat & three
            d = plsc.load_gather(dis_v, [node])
            xv = plsc.load_gather(xl_v, [node + comp * jnp.int32(NP)])
            s = pl.ds(off, 16)
            h_v[s] = h_v[s] + d * d * xv

    pltpu.sync_copy(h_v, out_hbm.at[wid])


def _head_body(hp_ref, b1_ref, wfc1_ref, bfc1_ref, wfc2_ref, bfc2_ref,
               out_ref, wp_ref):
    wp_ref[:NPG * H, :] = wfc1_ref[...]
    wp_ref[NPG * H:, :] = jnp.zeros((GPH - NPG * H, 32), jnp.float32)
    wsum = jnp.sum(wfc1_ref[...].reshape(NPG, H, 32), axis=0)
    beff = (jnp.dot(b1_ref[...], wsum, preferred_element_type=jnp.float32)
            + bfc1_ref[...])

    h = jnp.concatenate(
        [jnp.sum(hp_ref[:, g * GPH:(g + 1) * GPH], axis=0, keepdims=True)
         for g in range(B)], axis=0)
    a = jnp.dot(h, wp_ref[...], preferred_element_type=jnp.float32)
    a = jnp.maximum(a + beff, 0.0)
    out_ref[...] = jnp.dot(a, wfc2_ref[...],
                           preferred_element_type=jnp.float32) + bfc2_ref[...]


_head_call = pl.pallas_call(
    _head_body,
    out_shape=jax.ShapeDtypeStruct((B, 2), jnp.float32),
    scratch_shapes=[
        pltpu.VMEM((GPH, 32), jnp.float32),
    ],
)


def kernel(x, edge_index, edge_attr, y, adj, W1, b1, Wfc1, bfc1, Wfc2, bfc2):
    eflat = edge_index.reshape(2 * E)
    deg_p = _deg_kernel(eflat, edge_attr)
    xl0, xl1, xl2, xl3 = _lin_call(x, W1)
    dis = _dis_call(deg_p)
    h_p = _msg_kernel(eflat, edge_attr, xl0, xl1, xl2, xl3, dis)

    logits = _head_call(h_p, b1.reshape(1, H), Wfc1, bfc1.reshape(1, 32),
                        Wfc2, bfc2.reshape(1, 2))
    reg = jnp.zeros((1,), dtype=jnp.float32)
    return (logits, reg)
